# baseline (speedup 1.0000x reference)
import functools

import jax
import jax.numpy as jnp
from jax import lax
from jax.experimental import pallas as pl
from jax.experimental.pallas import tpu as pltpu
from jax.experimental.pallas import tpu_sc as plsc

K = 1024
D = 256
BETA = 0.25

TN = 1024


def _tc_body_common(x_ref, e2_ref, se2_ref, gold_ref,
                    inds_ref, loss_ref, corr_ref, quant_ref):
    x = x_ref[...]
    e2 = e2_ref[...]
    mm2 = lax.dot_general(x, e2, (((1,), (1,)), ((), ())),
                          preferred_element_type=jnp.float32)
    sx2 = jnp.sum(x * x, axis=1, keepdims=True)
    dist = sx2 + se2_ref[...] - mm2
    mind = jnp.min(dist, axis=1, keepdims=True)
    kiota = lax.broadcasted_iota(jnp.int32, (TN, K), 1)
    inds = jnp.min(jnp.where(dist == mind, kiota, K),
                   axis=1, keepdims=True)
    inds_ref[...] = inds
    lv = mind * (1.0 / D)
    loss_ref[...] = lv + BETA * lv
    corr_ref[0, 0, 0] = jnp.sum((gold_ref[...] == inds).astype(jnp.int32))
    if quant_ref is not None:
        onehot = (kiota == inds).astype(jnp.float32)
        quant_ref[...] = lax.dot_general(
            onehot, e2_ref[...], (((1,), (0,)), ((), ())),
            preferred_element_type=jnp.float32) * 0.5


def _tc_body_argmin(x_ref, e2_ref, se2_ref, gold_ref,
                    inds_ref, loss_ref, corr_ref):
    _tc_body_common(x_ref, e2_ref, se2_ref, gold_ref,
                    inds_ref, loss_ref, corr_ref, None)


def _tc_body_quant(x_ref, e2_ref, se2_ref, gold_ref,
                   inds_ref, loss_ref, corr_ref, quant_ref):
    _tc_body_common(x_ref, e2_ref, se2_ref, gold_ref,
                    inds_ref, loss_ref, corr_ref, quant_ref)


def _tc_call(flat, e2, se2, gold, with_quant, tile_lo, n_tiles):
    nh = n_tiles * TN
    out_specs = [
        pl.BlockSpec((TN, 1), lambda i: (i, 0)),
        pl.BlockSpec((TN, 1), lambda i: (i, 0)),
        pl.BlockSpec((1, 1, 1), lambda i: (i, 0, 0), memory_space=pltpu.SMEM),
    ]
    out_shape = [
        jax.ShapeDtypeStruct((nh, 1), jnp.int32),
        jax.ShapeDtypeStruct((nh, 1), jnp.float32),
        jax.ShapeDtypeStruct((n_tiles, 1, 1), jnp.int32),
    ]
    if with_quant:
        out_specs.append(pl.BlockSpec((TN, D), lambda i: (i, 0)))
        out_shape.append(jax.ShapeDtypeStruct((nh, D), jnp.float32))
    return pl.pallas_call(
        _tc_body_quant if with_quant else _tc_body_argmin,
        grid=(n_tiles,),
        in_specs=[
            pl.BlockSpec((TN, D), lambda i: (i + tile_lo, 0)),
            pl.BlockSpec((K, D), lambda i: (0, 0)),
            pl.BlockSpec((1, K), lambda i: (0, 0)),
            pl.BlockSpec((TN, 1), lambda i: (i + tile_lo, 0)),
        ],
        out_specs=out_specs,
        out_shape=out_shape,
    )(flat, e2, se2, gold)


_SC_CHUNK = 96
_SC_NBUF = 3


def _make_sc_gather(n):
    info = plsc.get_sparse_core_info()
    nw = info.num_cores * info.num_subcores
    n_per_w = n // nw
    n_chunks = n_per_w // _SC_CHUNK
    mesh = plsc.VectorSubcoreMesh(core_axis_name="c", subcore_axis_name="s")

    @functools.partial(
        pl.kernel, mesh=mesh,
        out_type=jax.ShapeDtypeStruct((n, D), jnp.float32),
        scratch_types=[
            pltpu.VMEM((n_chunks, _SC_CHUNK), jnp.int32),
            pltpu.VMEM((_SC_NBUF, _SC_CHUNK, D), jnp.float32),
            pltpu.SemaphoreType.DMA,
            pltpu.SemaphoreType.DMA,
        ],
    )
    def gather(table_hbm, idx_hbm, out_hbm, idx_v, rows_v, gsem, osem):
        wid = lax.axis_index("s") * info.num_cores + lax.axis_index("c")
        base = wid * n_per_w

        pltpu.sync_copy(idx_hbm.at[wid], idx_v)

        gd = [None] * n_chunks
        od = [None] * n_chunks
        for i in range(n_chunks):
            b = i % _SC_NBUF
            if i >= _SC_NBUF:
                od[i - _SC_NBUF].wait()
            gd[i] = pltpu.async_copy(table_hbm.at[idx_v.at[i]],
                                     rows_v.at[b], gsem)
            if i >= 1:
                gd[i - 1].wait()
                poff = base + (i - 1) * _SC_CHUNK
                od[i - 1] = pltpu.async_copy(
                    rows_v.at[(i - 1) % _SC_NBUF],
                    out_hbm.at[pl.ds(poff, _SC_CHUNK)], osem)
        gd[n_chunks - 1].wait()
        loff = base + (n_chunks - 1) * _SC_CHUNK
        od[n_chunks - 1] = pltpu.async_copy(
            rows_v.at[(n_chunks - 1) % _SC_NBUF],
            out_hbm.at[pl.ds(loff, _SC_CHUNK)], osem)
        for i in range(max(0, n_chunks - _SC_NBUF), n_chunks):
            if od[i] is not None:
                od[i].wait()

    return gather


def kernel(gold_encoding_inds, latents, epc, embedding_weight):
    b, t, d = latents.shape
    n = b * t
    na = n // 2
    flat = latents.reshape(n, d)
    se2 = jnp.sum(embedding_weight ** 2, axis=1)[None, :]
    gold = gold_encoding_inds.astype(jnp.int32)
    e2 = embedding_weight + embedding_weight

    indsA, lossA, corrA = _tc_call(flat, e2, se2, gold, False,
                                   0, na // TN)
    info = plsc.get_sparse_core_info()
    nw = info.num_cores * info.num_subcores
    idx3d = indsA.reshape(nw, (na // nw) // _SC_CHUNK, _SC_CHUNK)
    quantA = _make_sc_gather(na)(embedding_weight, idx3d)

    indsB, lossB, corrB, quantB = _tc_call(flat, e2, se2, gold, True,
                                           na // TN, (n - na) // TN)

    quant = jnp.concatenate([quantA, quantB], axis=0)
    inds = jnp.concatenate([indsA, indsB], axis=0).reshape(n)
    loss = jnp.concatenate([lossA, lossB], axis=0)
    quantized_latents = quant.reshape(b, t, d)
    vq_loss = loss.reshape(b, t)
    correct = jnp.sum(corrA) + jnp.sum(corrB)
    return (quantized_latents, vq_loss, inds.reshape(1, n), correct, n)


# speedup vs baseline: 1.3115x; 1.3115x over previous
import functools

import jax
import jax.numpy as jnp
from jax import lax
from jax.experimental import pallas as pl
from jax.experimental.pallas import tpu as pltpu
from jax.experimental.pallas import tpu_sc as plsc

K = 1024
D = 256
BETA = 0.25

TN = 1024


def _tc_body(x_ref, e2_ref, se2_ref, kio_ref, gold_ref,
             inds_ref, loss_ref, corr_ref):
    x = x_ref[...]
    e2 = e2_ref[...]
    mm2 = lax.dot_general(x, e2, (((1,), (1,)), ((), ())),
                          preferred_element_type=jnp.float32)
    sx2 = jnp.sum(x * x, axis=1, keepdims=True)
    dist = sx2 + se2_ref[...] - mm2
    mind = jnp.min(dist, axis=1, keepdims=True)
    inds = jnp.min(jnp.where(dist == mind, kio_ref[...], jnp.float32(K)),
                   axis=1, keepdims=True).astype(jnp.int32)
    inds_ref[...] = inds
    lv = mind * (1.0 / D)
    loss_ref[...] = lv + BETA * lv
    corr_ref[0, 0, 0] = jnp.sum((gold_ref[...] == inds).astype(jnp.int32))


def _tc_call(flat, e, se2, kio, gold, n_tiles):
    return pl.pallas_call(
        _tc_body,
        grid=(n_tiles,),
        in_specs=[
            pl.BlockSpec((TN, D), lambda i: (i, 0)),
            pl.BlockSpec((K, D), lambda i: (0, 0)),
            pl.BlockSpec((1, K), lambda i: (0, 0)),
            pl.BlockSpec((1, K), lambda i: (0, 0)),
            pl.BlockSpec((TN, 1), lambda i: (i, 0)),
        ],
        out_specs=[
            pl.BlockSpec((TN, 1), lambda i: (i, 0)),
            pl.BlockSpec((TN, 1), lambda i: (i, 0)),
            pl.BlockSpec((1, 1, 1), lambda i: (i, 0, 0), memory_space=pltpu.SMEM),
        ],
        out_shape=[
            jax.ShapeDtypeStruct((n_tiles * TN, 1), jnp.int32),
            jax.ShapeDtypeStruct((n_tiles * TN, 1), jnp.float32),
            jax.ShapeDtypeStruct((n_tiles, 1, 1), jnp.int32),
        ],
    )(flat, e, se2, kio, gold)


_SC_CHUNK = 128
_SC_NBUF = 3


def _make_sc_gather(n):
    info = plsc.get_sparse_core_info()
    nw = info.num_cores * info.num_subcores
    n_per_w = n // nw
    n_chunks = n_per_w // _SC_CHUNK
    mesh = plsc.VectorSubcoreMesh(core_axis_name="c", subcore_axis_name="s")

    @functools.partial(
        pl.kernel, mesh=mesh,
        out_type=jax.ShapeDtypeStruct((n, D), jnp.float32),
        scratch_types=[
            pltpu.VMEM((n_per_w // _SC_CHUNK, _SC_CHUNK), jnp.int32),
            pltpu.VMEM((_SC_NBUF, _SC_CHUNK, D), jnp.float32),
            pltpu.SemaphoreType.DMA,
            pltpu.SemaphoreType.DMA,
        ],
    )
    def gather(table_hbm, idx_hbm, out_hbm, idx_v, rows_v, gsem, osem):
        wid = lax.axis_index("s") * info.num_cores + lax.axis_index("c")
        base = wid * n_per_w

        pltpu.sync_copy(idx_hbm.at[wid], idx_v)

        gd = [None] * n_chunks
        od = [None] * n_chunks
        for i in range(n_chunks):
            b = i % _SC_NBUF
            if i >= _SC_NBUF:
                od[i - _SC_NBUF].wait()
            off = base + i * _SC_CHUNK
            gd[i] = pltpu.async_copy(table_hbm.at[idx_v.at[i]],
                                     rows_v.at[b], gsem)
            if i >= 1:
                gd[i - 1].wait()
                poff = base + (i - 1) * _SC_CHUNK
                od[i - 1] = pltpu.async_copy(
                    rows_v.at[(i - 1) % _SC_NBUF],
                    out_hbm.at[pl.ds(poff, _SC_CHUNK)], osem)
        gd[n_chunks - 1].wait()
        loff = base + (n_chunks - 1) * _SC_CHUNK
        od[n_chunks - 1] = pltpu.async_copy(
            rows_v.at[(n_chunks - 1) % _SC_NBUF],
            out_hbm.at[pl.ds(loff, _SC_CHUNK)], osem)
        for i in range(max(0, n_chunks - _SC_NBUF), n_chunks):
            if od[i] is not None:
                od[i].wait()

    return gather


def kernel(gold_encoding_inds, latents, epc, embedding_weight):
    b, t, d = latents.shape
    n = b * t
    flat = latents.reshape(n, d)
    se2 = jnp.sum(embedding_weight ** 2, axis=1)[None, :]
    kio = jnp.arange(K, dtype=jnp.float32)[None, :]
    gold = gold_encoding_inds.astype(jnp.int32)

    inds2d, loss2d, corr_part = _tc_call(flat, embedding_weight + embedding_weight,
                                         se2, kio, gold, n // TN)
    inds = inds2d.reshape(n)
    info = plsc.get_sparse_core_info()
    nw = info.num_cores * info.num_subcores
    idx3d = inds.reshape(nw, (n // nw) // _SC_CHUNK, _SC_CHUNK)
    quant = _make_sc_gather(n)(embedding_weight, idx3d)

    quantized_latents = quant.reshape(b, t, d)
    vq_loss = loss2d.reshape(b, t)
    correct = jnp.sum(corr_part)
    return (quantized_latents, vq_loss, inds.reshape(1, n), correct, n)


# speedup vs baseline: 1.4093x; 1.0746x over previous
import functools

import jax
import jax.numpy as jnp
from jax import lax
from jax.experimental import pallas as pl
from jax.experimental.pallas import tpu as pltpu
from jax.experimental.pallas import tpu_sc as plsc

K = 1024
D = 256
BETA = 0.25

TN = 2048


def _tc_body(x_ref, e2_ref, se2_ref, kio_ref, gold_ref,
             inds_ref, loss_ref, corr_ref):
    x = x_ref[...]
    e2 = e2_ref[...]
    mm2 = lax.dot_general(x, e2, (((1,), (1,)), ((), ())),
                          preferred_element_type=jnp.float32)
    sx2 = jnp.sum(x * x, axis=1, keepdims=True)
    dist = sx2 + se2_ref[...] - mm2
    mind = jnp.min(dist, axis=1, keepdims=True)
    inds = jnp.min(jnp.where(dist == mind, kio_ref[...], jnp.float32(K)),
                   axis=1, keepdims=True).astype(jnp.int32)
    inds_ref[...] = inds
    lv = mind * (1.0 / D)
    loss_ref[...] = lv + BETA * lv
    corr_ref[0, 0, 0] = jnp.sum((gold_ref[...] == inds).astype(jnp.int32))


def _tc_call(flat, e, se2, kio, gold, n_tiles):
    return pl.pallas_call(
        _tc_body,
        grid=(n_tiles,),
        in_specs=[
            pl.BlockSpec((TN, D), lambda i: (i, 0)),
            pl.BlockSpec((K, D), lambda i: (0, 0)),
            pl.BlockSpec((1, K), lambda i: (0, 0)),
            pl.BlockSpec((1, K), lambda i: (0, 0)),
            pl.BlockSpec((TN, 1), lambda i: (i, 0)),
        ],
        out_specs=[
            pl.BlockSpec((TN, 1), lambda i: (i, 0)),
            pl.BlockSpec((TN, 1), lambda i: (i, 0)),
            pl.BlockSpec((1, 1, 1), lambda i: (i, 0, 0), memory_space=pltpu.SMEM),
        ],
        out_shape=[
            jax.ShapeDtypeStruct((n_tiles * TN, 1), jnp.int32),
            jax.ShapeDtypeStruct((n_tiles * TN, 1), jnp.float32),
            jax.ShapeDtypeStruct((n_tiles, 1, 1), jnp.int32),
        ],
    )(flat, e, se2, kio, gold)


_SC_CHUNK = 128
_SC_NBUF = 3


def _make_sc_gather(n):
    info = plsc.get_sparse_core_info()
    nw = info.num_cores * info.num_subcores
    n_per_w = n // nw
    n_chunks = n_per_w // _SC_CHUNK
    mesh = plsc.VectorSubcoreMesh(core_axis_name="c", subcore_axis_name="s")

    @functools.partial(
        pl.kernel, mesh=mesh,
        out_type=jax.ShapeDtypeStruct((n, D), jnp.float32),
        scratch_types=[
            pltpu.VMEM((n_per_w // _SC_CHUNK, _SC_CHUNK), jnp.int32),
            pltpu.VMEM((_SC_NBUF, _SC_CHUNK, D), jnp.float32),
            pltpu.SemaphoreType.DMA,
            pltpu.SemaphoreType.DMA,
        ],
    )
    def gather(table_hbm, idx_hbm, out_hbm, idx_v, rows_v, gsem, osem):
        wid = lax.axis_index("s") * info.num_cores + lax.axis_index("c")
        base = wid * n_per_w

        pltpu.sync_copy(idx_hbm.at[wid], idx_v)

        gd = [None] * n_chunks
        od = [None] * n_chunks
        for i in range(n_chunks):
            b = i % _SC_NBUF
            if i >= _SC_NBUF:
                od[i - _SC_NBUF].wait()
            off = base + i * _SC_CHUNK
            gd[i] = pltpu.async_copy(table_hbm.at[idx_v.at[i]],
                                     rows_v.at[b], gsem)
            if i >= 1:
                gd[i - 1].wait()
                poff = base + (i - 1) * _SC_CHUNK
                od[i - 1] = pltpu.async_copy(
                    rows_v.at[(i - 1) % _SC_NBUF],
                    out_hbm.at[pl.ds(poff, _SC_CHUNK)], osem)
        gd[n_chunks - 1].wait()
        loff = base + (n_chunks - 1) * _SC_CHUNK
        od[n_chunks - 1] = pltpu.async_copy(
            rows_v.at[(n_chunks - 1) % _SC_NBUF],
            out_hbm.at[pl.ds(loff, _SC_CHUNK)], osem)
        for i in range(max(0, n_chunks - _SC_NBUF), n_chunks):
            if od[i] is not None:
                od[i].wait()

    return gather


def kernel(gold_encoding_inds, latents, epc, embedding_weight):
    b, t, d = latents.shape
    n = b * t
    flat = latents.reshape(n, d)
    se2 = jnp.sum(embedding_weight ** 2, axis=1)[None, :]
    kio = jnp.arange(K, dtype=jnp.float32)[None, :]
    gold = gold_encoding_inds.astype(jnp.int32)

    inds2d, loss2d, corr_part = _tc_call(flat, embedding_weight + embedding_weight,
                                         se2, kio, gold, n // TN)
    inds = inds2d.reshape(n)
    info = plsc.get_sparse_core_info()
    nw = info.num_cores * info.num_subcores
    idx3d = inds.reshape(nw, (n // nw) // _SC_CHUNK, _SC_CHUNK)
    quant = _make_sc_gather(n)(embedding_weight, idx3d)

    quantized_latents = quant.reshape(b, t, d)
    vq_loss = loss2d.reshape(b, t)
    correct = jnp.sum(corr_part)
    return (quantized_latents, vq_loss, inds.reshape(1, n), correct, n)


# speedup vs baseline: 1.4931x; 1.0595x over previous
import functools

import jax
import jax.numpy as jnp
from jax import lax
from jax.experimental import pallas as pl
from jax.experimental.pallas import tpu as pltpu
from jax.experimental.pallas import tpu_sc as plsc

K = 1024
D = 256
BETA = 0.25

TN = 4096


def _tc_body(x_ref, e2_ref, se2_ref, kio_ref, gold_ref,
             inds_ref, loss_ref, corr_ref):
    x = x_ref[...]
    e2 = e2_ref[...]
    mm2 = lax.dot_general(x, e2, (((1,), (1,)), ((), ())),
                          preferred_element_type=jnp.float32)
    sx2 = jnp.sum(x * x, axis=1, keepdims=True)
    dist = sx2 + se2_ref[...] - mm2
    mind = jnp.min(dist, axis=1, keepdims=True)
    inds = jnp.min(jnp.where(dist == mind, kio_ref[...], jnp.float32(K)),
                   axis=1, keepdims=True).astype(jnp.int32)
    inds_ref[...] = inds
    lv = mind * (1.0 / D)
    loss_ref[...] = lv + BETA * lv
    corr_ref[0, 0, 0] = jnp.sum((gold_ref[...] == inds).astype(jnp.int32))


def _tc_call(flat, e, se2, kio, gold, n_tiles):
    return pl.pallas_call(
        _tc_body,
        grid=(n_tiles,),
        in_specs=[
            pl.BlockSpec((TN, D), lambda i: (i, 0)),
            pl.BlockSpec((K, D), lambda i: (0, 0)),
            pl.BlockSpec((1, K), lambda i: (0, 0)),
            pl.BlockSpec((1, K), lambda i: (0, 0)),
            pl.BlockSpec((TN, 1), lambda i: (i, 0)),
        ],
        out_specs=[
            pl.BlockSpec((TN, 1), lambda i: (i, 0)),
            pl.BlockSpec((TN, 1), lambda i: (i, 0)),
            pl.BlockSpec((1, 1, 1), lambda i: (i, 0, 0), memory_space=pltpu.SMEM),
        ],
        out_shape=[
            jax.ShapeDtypeStruct((n_tiles * TN, 1), jnp.int32),
            jax.ShapeDtypeStruct((n_tiles * TN, 1), jnp.float32),
            jax.ShapeDtypeStruct((n_tiles, 1, 1), jnp.int32),
        ],
    )(flat, e, se2, kio, gold)


_SC_CHUNK = 128
_SC_NBUF = 3


def _make_sc_gather(n):
    info = plsc.get_sparse_core_info()
    nw = info.num_cores * info.num_subcores
    n_per_w = n // nw
    n_chunks = n_per_w // _SC_CHUNK
    mesh = plsc.VectorSubcoreMesh(core_axis_name="c", subcore_axis_name="s")

    @functools.partial(
        pl.kernel, mesh=mesh,
        out_type=jax.ShapeDtypeStruct((n, D), jnp.float32),
        scratch_types=[
            pltpu.VMEM((n_per_w // _SC_CHUNK, _SC_CHUNK), jnp.int32),
            pltpu.VMEM((_SC_NBUF, _SC_CHUNK, D), jnp.float32),
            pltpu.SemaphoreType.DMA,
            pltpu.SemaphoreType.DMA,
        ],
    )
    def gather(table_hbm, idx_hbm, out_hbm, idx_v, rows_v, gsem, osem):
        wid = lax.axis_index("s") * info.num_cores + lax.axis_index("c")
        base = wid * n_per_w

        pltpu.sync_copy(idx_hbm.at[wid], idx_v)

        gd = [None] * n_chunks
        od = [None] * n_chunks
        for i in range(n_chunks):
            b = i % _SC_NBUF
            if i >= _SC_NBUF:
                od[i - _SC_NBUF].wait()
            off = base + i * _SC_CHUNK
            gd[i] = pltpu.async_copy(table_hbm.at[idx_v.at[i]],
                                     rows_v.at[b], gsem)
            if i >= 1:
                gd[i - 1].wait()
                poff = base + (i - 1) * _SC_CHUNK
                od[i - 1] = pltpu.async_copy(
                    rows_v.at[(i - 1) % _SC_NBUF],
                    out_hbm.at[pl.ds(poff, _SC_CHUNK)], osem)
        gd[n_chunks - 1].wait()
        loff = base + (n_chunks - 1) * _SC_CHUNK
        od[n_chunks - 1] = pltpu.async_copy(
            rows_v.at[(n_chunks - 1) % _SC_NBUF],
            out_hbm.at[pl.ds(loff, _SC_CHUNK)], osem)
        for i in range(max(0, n_chunks - _SC_NBUF), n_chunks):
            if od[i] is not None:
                od[i].wait()

    return gather


def kernel(gold_encoding_inds, latents, epc, embedding_weight):
    b, t, d = latents.shape
    n = b * t
    flat = latents.reshape(n, d)
    se2 = jnp.sum(embedding_weight ** 2, axis=1)[None, :]
    kio = jnp.arange(K, dtype=jnp.float32)[None, :]
    gold = gold_encoding_inds.astype(jnp.int32)

    inds2d, loss2d, corr_part = _tc_call(flat, embedding_weight + embedding_weight,
                                         se2, kio, gold, n // TN)
    inds = inds2d.reshape(n)
    info = plsc.get_sparse_core_info()
    nw = info.num_cores * info.num_subcores
    idx3d = inds.reshape(nw, (n // nw) // _SC_CHUNK, _SC_CHUNK)
    quant = _make_sc_gather(n)(embedding_weight, idx3d)

    quantized_latents = quant.reshape(b, t, d)
    vq_loss = loss2d.reshape(b, t)
    correct = jnp.sum(corr_part)
    return (quantized_latents, vq_loss, inds.reshape(1, n), correct, n)
